# baseline (speedup 1.0000x reference)
import functools

import jax
import jax.numpy as jnp
from jax import lax
from jax.experimental import pallas as pl
from jax.experimental.pallas import tpu as pltpu
from jax.experimental.pallas import tpu_sc as plsc

F_DIM = 128
K_NEG = 65536
B = 1024

NUM_CORES = 2
NUM_SUBCORES = 16
NUM_WORKERS = NUM_CORES * NUM_SUBCORES
LANES = 16

ROW_BLKS = 4
COL_BLK = 128
ROWS_PER_BLK = F_DIM // ROW_BLKS
COL_BLKS = B // COL_BLK
LAB_PER_W = B // NUM_WORKERS

_mesh = plsc.VectorSubcoreMesh(core_axis_name="c", subcore_axis_name="s")


@functools.partial(
    pl.kernel,
    out_type=jax.ShapeDtypeStruct((1,), jnp.int32),
    mesh=_mesh,
    compiler_params=pltpu.CompilerParams(needs_layout_passes=False),
    scratch_types=[
        pltpu.VMEM((COL_BLK, F_DIM), jnp.float32),
        pltpu.VMEM((ROWS_PER_BLK, COL_BLK), jnp.float32),
        pltpu.VMEM((LAB_PER_W,), jnp.int32),
        pltpu.VMEM((1,), jnp.int32),
        pltpu.VMEM((LANES,), jnp.int32),
    ],
)
def _sc_update(keys_hbm, labels_hbm, ptr_hbm, buf_ref, lab_ref, ptr_out,
               keys_v, colbuf_v, lab_v, ptr_v, nptr_v):
    wid = lax.axis_index("s") * NUM_CORES + lax.axis_index("c")
    rb = lax.rem(wid, ROW_BLKS)
    cb = lax.div(wid, ROW_BLKS)
    f0 = rb * ROWS_PER_BLK
    j0 = cb * COL_BLK

    pltpu.sync_copy(ptr_hbm, ptr_v)
    zeros16 = jnp.zeros((LANES,), jnp.int32)
    pvec = plsc.load_gather(ptr_v, [zeros16])
    ptr_s = pl.multiple_of(jnp.max(jnp.clip(pvec, 0, K_NEG - B)), COL_BLK)

    pltpu.sync_copy(keys_hbm.at[pl.ds(j0, COL_BLK)], keys_v)

    def transpose_row(fr, carry):
        for h in range(COL_BLK // LANES):
            rows = lax.iota(jnp.int32, LANES) + h * LANES
            cols = jnp.full((LANES,), f0 + fr, jnp.int32)
            colbuf_v[fr, pl.ds(h * LANES, LANES)] = plsc.load_gather(
                keys_v, [rows, cols])
        return carry

    lax.fori_loop(0, ROWS_PER_BLK, transpose_row, 0)

    pltpu.sync_copy(
        colbuf_v,
        buf_ref.at[pl.ds(f0, ROWS_PER_BLK), pl.ds(ptr_s + j0, COL_BLK)])

    l0 = wid * LAB_PER_W
    pltpu.sync_copy(labels_hbm.at[pl.ds(l0, LAB_PER_W)], lab_v)
    pltpu.sync_copy(lab_v, lab_ref.at[pl.ds(ptr_s + l0, LAB_PER_W)])

    @pl.when(wid == 0)
    def _():
        nptr_v[...] = lax.rem(pvec + B, K_NEG)
        pltpu.sync_copy(nptr_v.at[pl.ds(0, 1)], ptr_out)


def kernel(keys, labels, buffer, mem_labels, ptr):
    buf_ref = jax.new_ref(buffer)
    lab_ref = jax.new_ref(mem_labels)
    new_ptr = _sc_update(keys, labels, ptr, buf_ref, lab_ref)
    return jax.freeze(buf_ref), jax.freeze(lab_ref), new_ptr
